# Initial kernel scaffold; baseline (speedup 1.0000x reference)
import jax
import jax.numpy as jnp
from jax.experimental import pallas as pl


def kernel(x, edge_index, W1, b1, W2, b2, W3, b3):
    raise NotImplementedError("write your pallas kernel here")


# speedup vs baseline: 9.5632x; 9.5632x over previous
import functools

import jax
import jax.numpy as jnp
from jax import lax
from jax.experimental import pallas as pl
from jax.experimental.pallas import tpu as pltpu
from jax.experimental.pallas import tpu_sc as plsc

N = 10000
E = 320000
D_IN = 128
D_HID = 256
D_OUT = 128

NC = 2
NS = 16
CH = 128
NCHUNK = 79
EPT = NCHUNK * CH
EPAD = NC * NS * EPT
NPAD = 10112
RPT = NPAD // NS
ZR = 64

_MESH = plsc.VectorSubcoreMesh(core_axis_name="c", subcore_axis_name="s")


def _zero_fill_vmem(ref, rows, cols):
    z = jnp.zeros((16,), jnp.float32)

    def fill(i, carry):
        for j in range(cols // 16):
            ref[i, pl.ds(j * 16, 16)] = z
        return carry

    lax.fori_loop(0, rows, fill, 0)


def _zero_acc_slice(acc, zero_v, base, width):
    nfull = RPT // ZR
    rem = RPT - nfull * ZR
    for k in range(nfull):
        pltpu.sync_copy(zero_v, acc.at[pl.ds(base + k * ZR, ZR)])
    if rem:
        pltpu.sync_copy(zero_v.at[pl.ds(0, rem)], acc.at[pl.ds(base + nfull * ZR, rem)])


def _make_deg_kernel():
    @functools.partial(
        pl.kernel,
        out_type=jax.ShapeDtypeStruct((NC, NPAD, 16), jnp.float32),
        mesh=_MESH,
        scratch_types=[
            pltpu.VMEM((NCHUNK, CH), jnp.int32),
            pltpu.VMEM((CH, 16), jnp.float32),
            pltpu.VMEM((ZR, 16), jnp.float32),
            pltpu.VMEM_SHARED((NPAD, 16), jnp.float32),
        ],
    )
    def deg_kernel(dst_hbm, out_hbm, dst_v, ones_v, zero_v, acc):
        c = lax.axis_index("c")
        s = lax.axis_index("s")
        wid = s * NC + c
        base = s * RPT
        pltpu.sync_copy(dst_hbm.at[wid], dst_v)
        one = jnp.full((16,), 1.0, jnp.float32)

        def fill(i, carry):
            ones_v[i, pl.ds(0, 16)] = one
            return carry

        lax.fori_loop(0, CH, fill, 0)
        _zero_fill_vmem(zero_v, ZR, 16)
        _zero_acc_slice(acc, zero_v, base, 16)
        plsc.subcore_barrier()

        def body(j, carry):
            pltpu.sync_copy(ones_v, acc.at[dst_v.at[j]], add=True)
            return carry

        lax.fori_loop(0, NCHUNK, body, 0)
        plsc.subcore_barrier()
        pltpu.sync_copy(acc.at[pl.ds(base, RPT)], out_hbm.at[c, pl.ds(base, RPT)])

    return deg_kernel


def _make_agg_kernel(ntab):
    @functools.partial(
        pl.kernel,
        out_type=[jax.ShapeDtypeStruct((NC, NPAD, 128), jnp.float32)] * ntab,
        mesh=_MESH,
        scratch_types=[
            pltpu.VMEM((NCHUNK, CH), jnp.int32),
            pltpu.VMEM((NCHUNK, CH), jnp.int32),
            pltpu.VMEM((CH, 128), jnp.float32),
            pltpu.VMEM((ZR, 128), jnp.float32),
            pltpu.VMEM_SHARED((NPAD, 128), jnp.float32),
            pltpu.SemaphoreType.DMA,
        ],
    )
    def agg_kernel(*refs):
        tabs = refs[:ntab]
        src_hbm = refs[ntab]
        dst_hbm = refs[ntab + 1]
        outs = refs[ntab + 2 : 2 * ntab + 2]
        src_v, dst_v, rows_v, zero_v, acc, sem = refs[2 * ntab + 2 :]
        c = lax.axis_index("c")
        s = lax.axis_index("s")
        wid = s * NC + c
        base = s * RPT
        pltpu.sync_copy(src_hbm.at[wid], src_v)
        pltpu.sync_copy(dst_hbm.at[wid], dst_v)
        _zero_fill_vmem(zero_v, ZR, 128)
        for t in range(ntab):
            _zero_acc_slice(acc, zero_v, base, 128)
            plsc.subcore_barrier()

            def body(j, carry, _t=t):
                pltpu.async_copy(tabs[_t].at[src_v.at[j]], rows_v, sem).wait()
                pltpu.sync_copy(rows_v, acc.at[dst_v.at[j]], add=True)
                return carry

            lax.fori_loop(0, NCHUNK, body, 0)
            plsc.subcore_barrier()
            pltpu.sync_copy(acc.at[pl.ds(base, RPT)], outs[t].at[c, pl.ds(base, RPT)])
            if t + 1 < ntab:
                plsc.subcore_barrier()

    return agg_kernel


RB = 1000
GRID = (N // RB,)

_row_spec128 = pl.BlockSpec((RB, 128), lambda i: (i, 0))
_acc_spec = pl.BlockSpec((NC, RB, 128), lambda i: (0, i, 0))
_deg_spec = pl.BlockSpec((NC, RB, 16), lambda i: (0, i, 0))


def _dinv_of(degp_ref):
    deg = degp_ref[0, :, 0:1] + degp_ref[1, :, 0:1] + 1.0
    return lax.rsqrt(deg)


def _scale_body(x_ref, degp_ref, o_ref):
    o_ref[...] = x_ref[...] * _dinv_of(degp_ref)


def _l1_body(sp_ref, g0_ref, degp_ref, w1_ref, b1_ref, oa_ref, ob_ref):
    dinv = _dinv_of(degp_ref)
    a0 = dinv * (sp_ref[0] + sp_ref[1] + g0_ref[...])
    h1 = jnp.dot(a0, w1_ref[...], preferred_element_type=jnp.float32) + b1_ref[...]
    g1 = dinv * jnp.maximum(h1, 0.0)
    oa_ref[...] = g1[:, :128]
    ob_ref[...] = g1[:, 128:]


def _l2_body(spa_ref, spb_ref, g1a_ref, g1b_ref, degp_ref, w2_ref, b2_ref, w3_ref, o_ref):
    dinv = _dinv_of(degp_ref)
    a1 = jnp.concatenate(
        [
            dinv * (spa_ref[0] + spa_ref[1] + g1a_ref[...]),
            dinv * (spb_ref[0] + spb_ref[1] + g1b_ref[...]),
        ],
        axis=1,
    )
    h2 = jnp.dot(a1, w2_ref[...], preferred_element_type=jnp.float32) + b2_ref[...]
    h2 = jnp.maximum(h2, 0.0)
    m = jnp.dot(h2, w3_ref[...], preferred_element_type=jnp.float32)
    o_ref[...] = dinv * m


def _l3_body(sp_ref, g2_ref, degp_ref, b3_ref, o_ref):
    dinv = _dinv_of(degp_ref)
    a2 = dinv * (sp_ref[0] + sp_ref[1] + g2_ref[...])
    o_ref[...] = jax.nn.sigmoid(a2 + b3_ref[...])


def _full_spec(shape):
    nd = len(shape)
    return pl.BlockSpec(shape, lambda i: (0,) * nd)


def kernel(x, edge_index, W1, b1, W2, b2, W3, b3):
    src = edge_index[0]
    dst = edge_index[1]
    pad = EPAD - E
    src_p = jnp.concatenate([src, jnp.zeros((pad,), jnp.int32)]).reshape(NC * NS, NCHUNK, CH)
    dst_p = jnp.concatenate([dst, jnp.full((pad,), N, jnp.int32)]).reshape(NC * NS, NCHUNK, CH)
    b1r = b1.reshape(1, D_HID)
    b2r = b2.reshape(1, D_HID)
    b3r = b3.reshape(1, D_OUT)

    degp = _make_deg_kernel()(dst_p)

    g0 = pl.pallas_call(
        _scale_body,
        grid=GRID,
        in_specs=[_row_spec128, _deg_spec],
        out_specs=_row_spec128,
        out_shape=jax.ShapeDtypeStruct((N, 128), jnp.float32),
    )(x, degp)

    (sp0,) = _make_agg_kernel(1)(g0, src_p, dst_p)

    g1a, g1b = pl.pallas_call(
        _l1_body,
        grid=GRID,
        in_specs=[
            _acc_spec,
            _row_spec128,
            _deg_spec,
            _full_spec((D_IN, D_HID)),
            _full_spec((1, D_HID)),
        ],
        out_specs=[_row_spec128, _row_spec128],
        out_shape=[
            jax.ShapeDtypeStruct((N, 128), jnp.float32),
            jax.ShapeDtypeStruct((N, 128), jnp.float32),
        ],
    )(sp0, g0, degp, W1, b1r)

    spa, spb = _make_agg_kernel(2)(g1a, g1b, src_p, dst_p)

    g2 = pl.pallas_call(
        _l2_body,
        grid=GRID,
        in_specs=[
            _acc_spec,
            _acc_spec,
            _row_spec128,
            _row_spec128,
            _deg_spec,
            _full_spec((D_HID, D_HID)),
            _full_spec((1, D_HID)),
            _full_spec((D_HID, D_OUT)),
        ],
        out_specs=_row_spec128,
        out_shape=jax.ShapeDtypeStruct((N, 128), jnp.float32),
    )(spa, spb, g1a, g1b, degp, W2, b2r, W3)

    (sp2,) = _make_agg_kernel(1)(g2, src_p, dst_p)

    out = pl.pallas_call(
        _l3_body,
        grid=GRID,
        in_specs=[_acc_spec, _row_spec128, _deg_spec, _full_spec((1, D_OUT))],
        out_specs=_row_spec128,
        out_shape=jax.ShapeDtypeStruct((N, 128), jnp.float32),
    )(sp2, g2, degp, b3r)

    return out
